# baseline (speedup 1.0000x reference)
import functools

import jax
import jax.numpy as jnp
from jax import lax
from jax.experimental import pallas as pl
from jax.experimental.pallas import tpu as pltpu
from jax.experimental.pallas import tpu_sc as plsc

N = 50000
D = 64
H = 32
L = 4
E = 50000
BK = 128
EPAD = 51200
NBT = EPAD // BK
NS = 16
NC = 2
NBW = NBT // NS
NPAD = 50176
CH = NPAD // NS
CH8 = CH // 8
CHA = N // NS
BN = 2000
GRID = N // BN


@functools.cache
def _get_deg_kernel():
  mesh = plsc.VectorSubcoreMesh(core_axis_name="c", subcore_axis_name="s",
                                num_cores=NC, num_subcores=NS)

  @functools.partial(
      pl.kernel,
      out_type=jax.ShapeDtypeStruct((NC, NPAD, 16), jnp.float32),
      mesh=mesh,
      compiler_params=pltpu.CompilerParams(use_tc_tiling_on_sc=False),
      scratch_types=[
          pltpu.VMEM((NBW, BK), jnp.int32),
          pltpu.VMEM((BK, 16), jnp.float32),
          pltpu.VMEM((CH8, 16), jnp.float32),
          pltpu.VMEM_SHARED((NPAD, 16), jnp.float32),
      ],
  )
  def _deg_kernel(cols_hbm, deg_hbm, coli, ones_b, zbuf, acc):
    c = lax.axis_index("c")
    s = lax.axis_index("s")
    pltpu.sync_copy(cols_hbm.at[c, pl.ds(s * NBW, NBW)], coli)

    one16 = jnp.full((16,), 1.0, jnp.float32)
    zero16 = jnp.zeros((16,), jnp.float32)

    def fill_ones(i, carry):
        ones_b[i] = one16
        return carry

    lax.fori_loop(0, BK, fill_ones, 0)

    def fill_zero(i, carry):
        zbuf[i] = zero16
        return carry

    lax.fori_loop(0, CH8, fill_zero, 0)
    for k in range(8):
        pltpu.sync_copy(zbuf, acc.at[pl.ds(s * CH + k * CH8, CH8)])
    plsc.subcore_barrier()

    def scat(j, carry):
        pltpu.sync_copy(ones_b, acc.at[coli.at[j]], add=True)
        return carry

    lax.fori_loop(0, NBW, scat, 0)
    plsc.subcore_barrier()
    pltpu.sync_copy(acc.at[pl.ds(s * CH, CH)], deg_hbm.at[c, pl.ds(s * CH, CH)])

  return _deg_kernel


@functools.cache
def _get_agg_kernel():
  mesh = plsc.VectorSubcoreMesh(core_axis_name="c", subcore_axis_name="s",
                                num_cores=NC, num_subcores=NS)

  @functools.partial(
      pl.kernel,
      out_type=(jax.ShapeDtypeStruct((N, H), jnp.float32),
                jax.ShapeDtypeStruct((N, H), jnp.float32),
                jax.ShapeDtypeStruct((N, H), jnp.float32),
                jax.ShapeDtypeStruct((N, H), jnp.float32)),
      mesh=mesh,
      compiler_params=pltpu.CompilerParams(use_tc_tiling_on_sc=False),
      scratch_types=[
          pltpu.VMEM((NBW, BK), jnp.int32),
          pltpu.VMEM((NBW, BK), jnp.int32),
          pltpu.VMEM((BK, H), jnp.float32),
          pltpu.VMEM((BK, H), jnp.float32),
          pltpu.VMEM_SHARED((NPAD, H), jnp.float32),
          pltpu.SemaphoreType.DMA,
          pltpu.SemaphoreType.DMA,
      ],
  )
  def _agg_kernel(vu0_hbm, vu1_hbm, vi0_hbm, vi1_hbm,
                  rows_ui_hbm, cols_ui_hbm, rows_iu_hbm, cols_iu_hbm,
                  itm0_hbm, itm1_hbm, usr0_hbm, usr1_hbm,
                  rowi, coli, gbuf0, gbuf1, acc, sem0, sem1):
    c = lax.axis_index("c")
    s = lax.axis_index("s")

    def run(v_hbm, out_hbm):
        pltpu.sync_copy(v_hbm.at[pl.ds(s * CHA, CHA)],
                        acc.at[pl.ds(s * CHA, CHA)])
        plsc.subcore_barrier()
        pltpu.sync_copy(v_hbm.at[rowi.at[0]], gbuf0)

        def body(j2, carry):
            b0 = j2 * 2
            d1 = pltpu.async_copy(v_hbm.at[rowi.at[b0 + 1]], gbuf1, sem1)
            pltpu.sync_copy(gbuf0, acc.at[coli.at[b0]], add=True)
            d1.wait()
            d2 = pltpu.async_copy(v_hbm.at[rowi.at[b0 + 2]], gbuf0, sem0)
            pltpu.sync_copy(gbuf1, acc.at[coli.at[b0 + 1]], add=True)
            d2.wait()
            return carry

        lax.fori_loop(0, (NBW - 1) // 2, body, 0)
        pltpu.sync_copy(gbuf0, acc.at[coli.at[NBW - 1]], add=True)
        plsc.subcore_barrier()
        pltpu.sync_copy(acc.at[pl.ds(s * CHA, CHA)],
                        out_hbm.at[pl.ds(s * CHA, CHA)])

    def rel(v0_hbm, v1_hbm, rows_hbm, cols_hbm, o0_hbm, o1_hbm):
        pltpu.sync_copy(rows_hbm.at[pl.ds(s * NBW, NBW)], rowi)
        pltpu.sync_copy(cols_hbm.at[pl.ds(s * NBW, NBW)], coli)

        @pl.when(c == 0)
        def _():
            run(v0_hbm, o0_hbm)

        @pl.when(c == 1)
        def _():
            run(v1_hbm, o1_hbm)

    rel(vu0_hbm, vu1_hbm, rows_ui_hbm, cols_ui_hbm, itm0_hbm, itm1_hbm)
    rel(vi0_hbm, vi1_hbm, rows_iu_hbm, cols_iu_hbm, usr0_hbm, usr1_hbm)

  return _agg_kernel


def _rep(shape):
    nd = len(shape)
    return pl.BlockSpec(shape, lambda i, _n=nd: (0,) * _n)


def _pre_body(xu3, xi3, eau3, eai3, w_ui0, w_ui1, w_iu0, w_iu1,
              bu0, bu1, bi0, bi1,
              xu_o, xi_o, eau_o, cu0_o, cu1_o, ci0_o, ci1_o):
    xu = jnp.mean(xu3[...], axis=1)
    xi = jnp.mean(xi3[...], axis=1)
    eau = jnp.mean(eau3[...], axis=1)
    eai = jnp.mean(eai3[...], axis=1)
    xu_o[...] = xu
    xi_o[...] = xi
    eau_o[...] = eau

    def lin(a, w, b):
        return jnp.dot(a, w[...], preferred_element_type=jnp.float32) + b[...]

    cu0_o[...] = lin(eau, w_ui0, bu0)
    cu1_o[...] = lin(eau, w_ui1, bu1)
    ci0_o[...] = lin(eai, w_iu0, bi0)
    ci1_o[...] = lin(eai, w_iu1, bi1)


def _pre_call(xu3, xi3, eau3, eai3, ws, bs):
    big = pl.BlockSpec((BN, L, D), lambda i: (i, 0, 0))
    nd = jax.ShapeDtypeStruct((N, D), jnp.float32)
    return pl.pallas_call(
        _pre_body,
        grid=(GRID,),
        in_specs=[big] * 4 + [_rep((D, D))] * 4 + [_rep((1, D))] * 4,
        out_specs=[pl.BlockSpec((BN, D), lambda i: (i, 0))] * 7,
        out_shape=[nd] * 7,
    )(xu3, xi3, eau3, eai3, *ws, *bs)


def _v_body(halves, *refs):
    if halves:
        (xua, xub, xia, xib, w_ui, w_iu, cu, ci, du, di,
         vu0_o, vu1_o, vi0_o, vi1_o) = refs
        xu = jnp.concatenate([xua[...], xub[...]], axis=1)
        xi = jnp.concatenate([xia[...], xib[...]], axis=1)
    else:
        (xur, xir, w_ui, w_iu, cu, ci, du, di,
         vu0_o, vu1_o, vi0_o, vi1_o) = refs
        xu = xur[...]
        xi = xir[...]
    yu = lax.rsqrt(du[...] + 1.0) * (
        jnp.dot(xu, w_ui[...], preferred_element_type=jnp.float32) + cu[...])
    yi = lax.rsqrt(di[...] + 1.0) * (
        jnp.dot(xi, w_iu[...], preferred_element_type=jnp.float32) + ci[...])
    vu0_o[...] = yu[:, :H]
    vu1_o[...] = yu[:, H:]
    vi0_o[...] = yi[:, :H]
    vi1_o[...] = yi[:, H:]


def _v_call(halves, xs, w_ui, w_iu, cu, ci, du, di):
    blk = pl.BlockSpec((BN, D), lambda i: (i, 0))
    half = pl.BlockSpec((BN, H), lambda i: (i, 0))
    deg = pl.BlockSpec((BN, 1), lambda i: (i, 0))
    nh = jax.ShapeDtypeStruct((N, H), jnp.float32)
    x_specs = [half] * 4 if halves else [blk] * 2
    return pl.pallas_call(
        functools.partial(_v_body, halves),
        grid=(GRID,),
        in_specs=x_specs + [_rep((D, D))] * 2 + [blk] * 2 + [deg] * 2,
        out_specs=[half] * 4,
        out_shape=[nh] * 4,
    )(*xs, w_ui, w_iu, cu, ci, du, di)


def _fin_body(xu, xi, u1a, u1b, u2a, u2b, i1a, i1b, i2a, i2b, eau,
              wf1, bf1, wf2r, bf2, out):
    u = (xu[...]
         + jnp.concatenate([u1a[...], u1b[...]], axis=1)
         + jnp.concatenate([u2a[...], u2b[...]], axis=1))
    it = (xi[...]
          + jnp.concatenate([i1a[...], i1b[...]], axis=1)
          + jnp.concatenate([i2a[...], i2b[...]], axis=1))
    sv = (u + it) * (1.0 / 3.0) + 4.0 * eau[...]
    h = jnp.dot(sv, wf1[...], preferred_element_type=jnp.float32) + bf1[...]
    h = jnp.where(h >= 0, h, 0.01 * h)
    out[...] = jnp.sum(h * wf2r[...], axis=1, keepdims=True) + bf2[...]


def kernel(x_user, x_item, edge_attr_ui, edge_attr_iu,
           edge_index_ui, edge_index_iu,
           W1_ui0, b1_ui0, W2_ui0, b2_ui0, W3_ui0, b3_ui0,
           W1_iu0, b1_iu0, W2_iu0, b2_iu0, W3_iu0, b3_iu0,
           W1_ui1, b1_ui1, W2_ui1, b2_ui1, W3_ui1, b3_ui1,
           W1_iu1, b1_iu1, W2_iu1, b2_iu1, W3_iu1, b3_iu1,
           Wf1, bf1, Wf2, bf2):
    ei_ui = edge_index_ui.astype(jnp.int32)
    ei_iu = edge_index_iu.astype(jnp.int32)
    pad_r = jnp.zeros((EPAD - E,), jnp.int32)
    pad_c = jnp.full((EPAD - E,), N, jnp.int32)
    rows_ui = jnp.concatenate([ei_ui[0], pad_r]).reshape(NBT, BK)
    cols_ui = jnp.concatenate([ei_ui[1], pad_c]).reshape(NBT, BK)
    rows_iu = jnp.concatenate([ei_iu[0], pad_r]).reshape(NBT, BK)
    cols_iu = jnp.concatenate([ei_iu[1], pad_c]).reshape(NBT, BK)
    cols2 = jnp.stack([cols_ui, cols_iu])

    degc = _get_deg_kernel()(cols2)
    du = degc[0, :N, 0:1]
    di = degc[1, :N, 0:1]

    bsu0 = (b1_ui0 + b3_ui0).reshape(1, D)
    bsu1 = (b1_ui1 + b3_ui1).reshape(1, D)
    bsi0 = (b1_iu0 + b3_iu0).reshape(1, D)
    bsi1 = (b1_iu1 + b3_iu1).reshape(1, D)
    xu, xi, eau, cu0, cu1, ci0, ci1 = _pre_call(
        x_user, x_item, edge_attr_ui, edge_attr_iu,
        (W3_ui0, W3_ui1, W3_iu0, W3_iu1), (bsu0, bsu1, bsi0, bsi1))

    agg = _get_agg_kernel()
    vu0, vu1, vi0, vi1 = _v_call(False, (xu, xi), W1_ui0, W1_iu0,
                                 cu0, ci0, du, di)
    it1a, it1b, us1a, us1b = agg(vu0, vu1, vi0, vi1,
                                 rows_ui, cols_ui, rows_iu, cols_iu)

    vu0b, vu1b, vi0b, vi1b = _v_call(True, (us1a, us1b, it1a, it1b),
                                     W1_ui1, W1_iu1, cu1, ci1, du, di)
    it2a, it2b, us2a, us2b = agg(vu0b, vu1b, vi0b, vi1b,
                                 rows_ui, cols_ui, rows_iu, cols_iu)

    blk = pl.BlockSpec((BN, D), lambda i: (i, 0))
    half = pl.BlockSpec((BN, H), lambda i: (i, 0))
    logits = pl.pallas_call(
        _fin_body,
        grid=(GRID,),
        in_specs=[blk, blk] + [half] * 8 + [blk, _rep((D, D)),
                                            _rep((1, D)), _rep((1, D)),
                                            _rep((1, 1))],
        out_specs=pl.BlockSpec((BN, 1), lambda i: (i, 0)),
        out_shape=jax.ShapeDtypeStruct((N, 1), jnp.float32),
    )(xu, xi, us1a, us1b, us2a, us2b, it1a, it1b, it2a, it2b, eau,
      Wf1, bf1.reshape(1, D), Wf2.reshape(1, D), bf2.reshape(1, 1))
    return logits


# speedup vs baseline: 2.2334x; 2.2334x over previous
import functools

import jax
import jax.numpy as jnp
from jax import lax
from jax.experimental import pallas as pl
from jax.experimental.pallas import tpu as pltpu
from jax.experimental.pallas import tpu_sc as plsc

N = 50000
D = 64
H = 32
L = 4
E = 50000
BK = 128
EPAD = 51200
NBT = EPAD // BK
NS = 16
NC = 2
NBW = NBT // NS
NPAD = 50176
CH = NPAD // NS
CH8 = CH // 8
CHA = N // NS
ZR = 448
BN = 2000
GRID = N // BN
BT = 2048
GRIDT = -(-N // BT)


@functools.cache
def _get_deg_kernel():
  mesh = plsc.VectorSubcoreMesh(core_axis_name="c", subcore_axis_name="s",
                                num_cores=NC, num_subcores=NS)

  @functools.partial(
      pl.kernel,
      out_type=jax.ShapeDtypeStruct((NC, NPAD, 16), jnp.float32),
      mesh=mesh,
      compiler_params=pltpu.CompilerParams(use_tc_tiling_on_sc=False),
      scratch_types=[
          pltpu.VMEM((NBW, BK), jnp.int32),
          pltpu.VMEM((BK, 16), jnp.float32),
          pltpu.VMEM((CH8, 16), jnp.float32),
          pltpu.VMEM_SHARED((NPAD, 16), jnp.float32),
      ],
  )
  def _deg_kernel(cols_hbm, deg_hbm, coli, ones_b, zbuf, acc):
    c = lax.axis_index("c")
    s = lax.axis_index("s")
    pltpu.sync_copy(cols_hbm.at[c, pl.ds(s * NBW, NBW)], coli)

    one16 = jnp.full((16,), 1.0, jnp.float32)
    zero16 = jnp.zeros((16,), jnp.float32)

    def fill_ones(i, carry):
        ones_b[i] = one16
        return carry

    lax.fori_loop(0, BK, fill_ones, 0)

    def fill_zero(i, carry):
        zbuf[i] = zero16
        return carry

    lax.fori_loop(0, CH8, fill_zero, 0)
    for k in range(8):
        pltpu.sync_copy(zbuf, acc.at[pl.ds(s * CH + k * CH8, CH8)])
    plsc.subcore_barrier()

    def scat(j, carry):
        pltpu.sync_copy(ones_b, acc.at[coli.at[j]], add=True)
        return carry

    lax.fori_loop(0, NBW, scat, 0)
    plsc.subcore_barrier()
    pltpu.sync_copy(acc.at[pl.ds(s * CH, CH)], deg_hbm.at[c, pl.ds(s * CH, CH)])

  return _deg_kernel


@functools.cache
def _get_agg_kernel():
  mesh = plsc.VectorSubcoreMesh(core_axis_name="c", subcore_axis_name="s",
                                num_cores=NC, num_subcores=NS)

  @functools.partial(
      pl.kernel,
      out_type=jax.ShapeDtypeStruct((N, 2 * D), jnp.float32),
      mesh=mesh,
      compiler_params=pltpu.CompilerParams(use_tc_tiling_on_sc=False),
      scratch_types=[
          pltpu.VMEM((NBW, BK), jnp.int32),
          pltpu.VMEM((NBW, BK), jnp.int32),
          pltpu.VMEM((BK, H), jnp.float32),
          pltpu.VMEM((BK, H), jnp.float32),
          pltpu.VMEM((ZR, H), jnp.float32),
          pltpu.VMEM_SHARED((NPAD, H), jnp.float32),
          pltpu.SemaphoreType.DMA,
          pltpu.SemaphoreType.DMA,
      ],
  )
  def _agg_kernel(v2_hbm, rows_ui_hbm, cols_ui_hbm,
                  rows_iu_hbm, cols_iu_hbm, out_hbm,
                  rowi, coli, gbuf0, gbuf1, zbuf, acc, sem0, sem1):
    c = lax.axis_index("c")
    s = lax.axis_index("s")

    zero16 = jnp.zeros((16,), jnp.float32)

    def fill_zero(t, carry):
        zbuf[t // 2, pl.ds((t % 2) * 16, 16)] = zero16
        return carry

    lax.fori_loop(0, ZR * 2, fill_zero, 0)

    def run(vslot, oslot):
        for k in range(CH // ZR):
            pltpu.sync_copy(zbuf, acc.at[pl.ds(s * CH + k * ZR, ZR)])

        def scale(t, carry):
            j = t // 8
            tt = (t % 8) * 16
            vec = rowi[j, pl.ds(tt, 16)]
            rowi[j, pl.ds(tt, 16)] = vec * 4 + vslot
            return carry

        lax.fori_loop(0, NBW * 8, scale, 0)
        plsc.subcore_barrier()
        pltpu.sync_copy(v2_hbm.at[rowi.at[0]], gbuf0)

        def body(j2, carry):
            b0 = j2 * 2
            d1 = pltpu.async_copy(v2_hbm.at[rowi.at[b0 + 1]], gbuf1, sem1)
            pltpu.sync_copy(gbuf0, acc.at[coli.at[b0]], add=True)
            d1.wait()
            d2 = pltpu.async_copy(v2_hbm.at[rowi.at[b0 + 2]], gbuf0, sem0)
            pltpu.sync_copy(gbuf1, acc.at[coli.at[b0 + 1]], add=True)
            d2.wait()
            return carry

        lax.fori_loop(0, (NBW - 1) // 2, body, 0)
        pltpu.sync_copy(gbuf0, acc.at[coli.at[NBW - 1]], add=True)
        plsc.subcore_barrier()
        pltpu.sync_copy(acc.at[pl.ds(s * CHA, CHA)],
                        out_hbm.at[pl.ds(s * CHA, CHA), pl.ds(oslot * H, H)])

    def rel(rows_hbm, cols_hbm, vbase, obase):
        pltpu.sync_copy(rows_hbm.at[pl.ds(s * NBW, NBW)], rowi)
        pltpu.sync_copy(cols_hbm.at[pl.ds(s * NBW, NBW)], coli)
        run(vbase + c, obase + c)

    rel(rows_ui_hbm, cols_ui_hbm, 0, 2)
    rel(rows_iu_hbm, cols_iu_hbm, 2, 0)

  return _agg_kernel


def _rep(shape):
    nd = len(shape)
    return pl.BlockSpec(shape, lambda i, _n=nd: (0,) * _n)


def _pre_body(xu3, xi3, eau3, eai3, w_ui0, w_ui1, w_iu0, w_iu1,
              bu0, bu1, bi0, bi1,
              xu_o, xi_o, eau_o, cu0_o, cu1_o, ci0_o, ci1_o):
    xu = jnp.transpose(jnp.mean(xu3[...], axis=0))
    xi = jnp.transpose(jnp.mean(xi3[...], axis=0))
    eau = jnp.transpose(jnp.mean(eau3[...], axis=0))
    eai = jnp.transpose(jnp.mean(eai3[...], axis=0))
    xu_o[...] = xu
    xi_o[...] = xi
    eau_o[...] = eau

    def lin(a, w, b):
        return jnp.dot(a, w[...], preferred_element_type=jnp.float32) + b[...]

    cu0_o[...] = lin(eau, w_ui0, bu0)
    cu1_o[...] = lin(eau, w_ui1, bu1)
    ci0_o[...] = lin(eai, w_iu0, bi0)
    ci1_o[...] = lin(eai, w_iu1, bi1)


def _pre_call(xu3t, xi3t, eau3t, eai3t, ws, bs):
    big = pl.BlockSpec((L, D, BT), lambda i: (0, 0, i))
    nd = jax.ShapeDtypeStruct((N, D), jnp.float32)
    return pl.pallas_call(
        _pre_body,
        grid=(GRIDT,),
        in_specs=[big] * 4 + [_rep((D, D))] * 4 + [_rep((1, D))] * 4,
        out_specs=[pl.BlockSpec((BT, D), lambda i: (i, 0))] * 7,
        out_shape=[nd] * 7,
    )(xu3t, xi3t, eau3t, eai3t, *ws, *bs)


def _v_body(packed, *refs):
    if packed:
        (pk, vprev, w_ui, w_iu, cu, ci, du, di, vp_o) = refs
        xu = pk[:, :D] + vprev[:, D:]
        xi = pk[:, D:] + vprev[:, :D]
    else:
        (xur, xir, w_ui, w_iu, cu, ci, du, di, vp_o) = refs
        xu = xur[...]
        xi = xir[...]
    yu = lax.rsqrt(du[...] + 1.0) * (
        jnp.dot(xu, w_ui[...], preferred_element_type=jnp.float32) + cu[...])
    yi = lax.rsqrt(di[...] + 1.0) * (
        jnp.dot(xi, w_iu[...], preferred_element_type=jnp.float32) + ci[...])
    vp_o[...] = jnp.concatenate([yu, yi], axis=1)


def _v_call(packed, xs, w_ui, w_iu, cu, ci, du, di):
    blk = pl.BlockSpec((BN, D), lambda i: (i, 0))
    pk = pl.BlockSpec((BN, 2 * D), lambda i: (i, 0))
    deg = pl.BlockSpec((BN, 1), lambda i: (i, 0))
    x_specs = [pk, pk] if packed else [blk] * 2
    return pl.pallas_call(
        functools.partial(_v_body, packed),
        grid=(GRID,),
        in_specs=x_specs + [_rep((D, D))] * 2 + [blk] * 2 + [deg] * 2,
        out_specs=pk,
        out_shape=jax.ShapeDtypeStruct((N, 2 * D), jnp.float32),
    )(*xs, w_ui, w_iu, cu, ci, du, di)


def _fin_body(xu, xi, p1, p2, v0, v1, eau, wf1, bf1, wf2r, bf2, out):
    u = xu[...] + p1[:, :D] + v0[:, D:] + p2[:, :D] + v1[:, D:]
    it = xi[...] + p1[:, D:] + v0[:, :D] + p2[:, D:] + v1[:, :D]
    sv = (u + it) * (1.0 / 3.0) + 4.0 * eau[...]
    h = jnp.dot(sv, wf1[...], preferred_element_type=jnp.float32) + bf1[...]
    h = jnp.where(h >= 0, h, 0.01 * h)
    out[...] = jnp.sum(h * wf2r[...], axis=1, keepdims=True) + bf2[...]


def kernel(x_user, x_item, edge_attr_ui, edge_attr_iu,
           edge_index_ui, edge_index_iu,
           W1_ui0, b1_ui0, W2_ui0, b2_ui0, W3_ui0, b3_ui0,
           W1_iu0, b1_iu0, W2_iu0, b2_iu0, W3_iu0, b3_iu0,
           W1_ui1, b1_ui1, W2_ui1, b2_ui1, W3_ui1, b3_ui1,
           W1_iu1, b1_iu1, W2_iu1, b2_iu1, W3_iu1, b3_iu1,
           Wf1, bf1, Wf2, bf2):
    ei_ui = edge_index_ui.astype(jnp.int32)
    ei_iu = edge_index_iu.astype(jnp.int32)
    pad_r = jnp.zeros((EPAD - E,), jnp.int32)
    pad_c = jnp.full((EPAD - E,), N, jnp.int32)
    rows_ui = jnp.concatenate([ei_ui[0], pad_r]).reshape(NBT, BK)
    cols_ui = jnp.concatenate([ei_ui[1], pad_c]).reshape(NBT, BK)
    rows_iu = jnp.concatenate([ei_iu[0], pad_r]).reshape(NBT, BK)
    cols_iu = jnp.concatenate([ei_iu[1], pad_c]).reshape(NBT, BK)
    cols2 = jnp.stack([cols_ui, cols_iu])

    degc = _get_deg_kernel()(cols2)
    du = degc[0, :N, 0:1]
    di = degc[1, :N, 0:1]

    bsu0 = (b1_ui0 + b3_ui0).reshape(1, D)
    bsu1 = (b1_ui1 + b3_ui1).reshape(1, D)
    bsi0 = (b1_iu0 + b3_iu0).reshape(1, D)
    bsi1 = (b1_iu1 + b3_iu1).reshape(1, D)
    tr = lambda a: jnp.transpose(a, (1, 2, 0))
    xu, xi, eau, cu0, cu1, ci0, ci1 = _pre_call(
        tr(x_user), tr(x_item), tr(edge_attr_ui), tr(edge_attr_iu),
        (W3_ui0, W3_ui1, W3_iu0, W3_iu1), (bsu0, bsu1, bsi0, bsi1))

    agg = _get_agg_kernel()

    def agg_call(vpack):
        v2 = vpack.reshape(4 * N, H)
        return agg(v2, rows_ui, cols_ui, rows_iu, cols_iu)

    vp0 = _v_call(False, (xu, xi), W1_ui0, W1_iu0, cu0, ci0, du, di)
    pk1 = agg_call(vp0)

    vp1 = _v_call(True, (pk1, vp0), W1_ui1, W1_iu1, cu1, ci1, du, di)
    pk2 = agg_call(vp1)

    blk = pl.BlockSpec((BN, D), lambda i: (i, 0))
    pk = pl.BlockSpec((BN, 2 * D), lambda i: (i, 0))
    logits = pl.pallas_call(
        _fin_body,
        grid=(GRID,),
        in_specs=[blk, blk, pk, pk, pk, pk, blk, _rep((D, D)),
                  _rep((1, D)), _rep((1, D)), _rep((1, 1))],
        out_specs=pl.BlockSpec((BN, 1), lambda i: (i, 0)),
        out_shape=jax.ShapeDtypeStruct((N, 1), jnp.float32),
    )(xu, xi, pk1, pk2, vp0, vp1, eau,
      Wf1, bf1.reshape(1, D), Wf2.reshape(1, D), bf2.reshape(1, 1))
    return logits


# speedup vs baseline: 2.2529x; 1.0087x over previous
import functools

import jax
import jax.numpy as jnp
from jax import lax
from jax.experimental import pallas as pl
from jax.experimental.pallas import tpu as pltpu
from jax.experimental.pallas import tpu_sc as plsc

N = 50000
D = 64
H = 32
L = 4
E = 50000
BK = 128
EPAD = 51200
NBT = EPAD // BK
NS = 16
NC = 2
NBW = NBT // NS
NPAD = 50176
CH = NPAD // NS
CH8 = CH // 8
CHA = N // NS
ZR = 448
BN = 2000
GRID = N // BN
BT = 2048
GRIDT = -(-N // BT)


@functools.cache
def _get_deg_kernel():
  mesh = plsc.VectorSubcoreMesh(core_axis_name="c", subcore_axis_name="s",
                                num_cores=NC, num_subcores=NS)

  @functools.partial(
      pl.kernel,
      out_type=jax.ShapeDtypeStruct((NC, NPAD, 16), jnp.float32),
      mesh=mesh,
      compiler_params=pltpu.CompilerParams(use_tc_tiling_on_sc=False),
      scratch_types=[
          pltpu.VMEM((NBW, BK), jnp.int32),
          pltpu.VMEM((BK, 16), jnp.float32),
          pltpu.VMEM((CH8, 16), jnp.float32),
          pltpu.VMEM_SHARED((NPAD, 16), jnp.float32),
      ],
  )
  def _deg_kernel(cols_hbm, deg_hbm, coli, ones_b, zbuf, acc):
    c = lax.axis_index("c")
    s = lax.axis_index("s")
    pltpu.sync_copy(cols_hbm.at[c, pl.ds(s * NBW, NBW)], coli)

    one16 = jnp.full((16,), 1.0, jnp.float32)
    zero16 = jnp.zeros((16,), jnp.float32)

    def fill_ones(i, carry):
        ones_b[i] = one16
        return carry

    lax.fori_loop(0, BK, fill_ones, 0)

    def fill_zero(i, carry):
        zbuf[i] = zero16
        return carry

    lax.fori_loop(0, CH8, fill_zero, 0)
    for k in range(8):
        pltpu.sync_copy(zbuf, acc.at[pl.ds(s * CH + k * CH8, CH8)])
    plsc.subcore_barrier()

    def scat(j, carry):
        pltpu.sync_copy(ones_b, acc.at[coli.at[j]], add=True)
        return carry

    lax.fori_loop(0, NBW, scat, 0)
    plsc.subcore_barrier()
    pltpu.sync_copy(acc.at[pl.ds(s * CH, CH)], deg_hbm.at[c, pl.ds(s * CH, CH)])

  return _deg_kernel


@functools.cache
def _get_agg_kernel():
  mesh = plsc.VectorSubcoreMesh(core_axis_name="c", subcore_axis_name="s",
                                num_cores=NC, num_subcores=NS)

  @functools.partial(
      pl.kernel,
      out_type=jax.ShapeDtypeStruct((N, 2 * D), jnp.float32),
      mesh=mesh,
      compiler_params=pltpu.CompilerParams(use_tc_tiling_on_sc=False),
      scratch_types=[
          pltpu.VMEM((NBW, BK), jnp.int32),
          pltpu.VMEM((NBW, BK), jnp.int32),
          pltpu.VMEM((BK, H), jnp.float32),
          pltpu.VMEM((BK, H), jnp.float32),
          pltpu.VMEM((ZR, H), jnp.float32),
          pltpu.VMEM_SHARED((NPAD, H), jnp.float32),
          pltpu.SemaphoreType.DMA,
          pltpu.SemaphoreType.DMA,
      ],
  )
  def _agg_kernel(v2_hbm, rows_ui_hbm, cols_ui_hbm,
                  rows_iu_hbm, cols_iu_hbm, out_hbm,
                  rowi, coli, gbuf0, gbuf1, zbuf, acc, sem0, sem1):
    c = lax.axis_index("c")
    s = lax.axis_index("s")

    zero16 = jnp.zeros((16,), jnp.float32)

    def fill_zero(t, carry):
        zbuf[t // 2, pl.ds((t % 2) * 16, 16)] = zero16
        return carry

    lax.fori_loop(0, ZR * 2, fill_zero, 0)

    def run(vslot, oslot):
        zd = [pltpu.async_copy(zbuf, acc.at[pl.ds(s * CH + k * ZR, ZR)], sem0)
              for k in range(CH // ZR)]

        def scale(t, carry):
            j = t // 8
            tt = (t % 8) * 16
            vec = rowi[j, pl.ds(tt, 16)]
            rowi[j, pl.ds(tt, 16)] = vec * 4 + vslot
            return carry

        lax.fori_loop(0, NBW * 8, scale, 0)
        for d in zd:
            d.wait()
        plsc.subcore_barrier()
        pltpu.sync_copy(v2_hbm.at[rowi.at[0]], gbuf0)

        def body(j2, carry):
            b0 = j2 * 2
            d1 = pltpu.async_copy(v2_hbm.at[rowi.at[b0 + 1]], gbuf1, sem1)
            pltpu.sync_copy(gbuf0, acc.at[coli.at[b0]], add=True)
            d1.wait()
            d2 = pltpu.async_copy(v2_hbm.at[rowi.at[b0 + 2]], gbuf0, sem0)
            pltpu.sync_copy(gbuf1, acc.at[coli.at[b0 + 1]], add=True)
            d2.wait()
            return carry

        lax.fori_loop(0, (NBW - 1) // 2, body, 0)
        pltpu.sync_copy(gbuf0, acc.at[coli.at[NBW - 1]], add=True)
        plsc.subcore_barrier()
        pltpu.sync_copy(acc.at[pl.ds(s * CHA, CHA)],
                        out_hbm.at[pl.ds(s * CHA, CHA), pl.ds(oslot * H, H)])

    def rel(rows_hbm, cols_hbm, vbase, obase):
        pltpu.sync_copy(rows_hbm.at[pl.ds(s * NBW, NBW)], rowi)
        pltpu.sync_copy(cols_hbm.at[pl.ds(s * NBW, NBW)], coli)
        run(vbase + c, obase + c)

    rel(rows_ui_hbm, cols_ui_hbm, 0, 2)
    rel(rows_iu_hbm, cols_iu_hbm, 2, 0)

  return _agg_kernel


def _rep(shape):
    nd = len(shape)
    return pl.BlockSpec(shape, lambda i, _n=nd: (0,) * _n)


def _pre_body(xu3, xi3, eau3, eai3, w_ui0, w_ui1, w_iu0, w_iu1,
              bu0, bu1, bi0, bi1,
              xu_o, xi_o, eau_o, cu0_o, cu1_o, ci0_o, ci1_o):
    xu = jnp.transpose(jnp.mean(xu3[...], axis=0))
    xi = jnp.transpose(jnp.mean(xi3[...], axis=0))
    eau = jnp.transpose(jnp.mean(eau3[...], axis=0))
    eai = jnp.transpose(jnp.mean(eai3[...], axis=0))
    xu_o[...] = xu
    xi_o[...] = xi
    eau_o[...] = eau

    def lin(a, w, b):
        return jnp.dot(a, w[...], preferred_element_type=jnp.float32) + b[...]

    cu0_o[...] = lin(eau, w_ui0, bu0)
    cu1_o[...] = lin(eau, w_ui1, bu1)
    ci0_o[...] = lin(eai, w_iu0, bi0)
    ci1_o[...] = lin(eai, w_iu1, bi1)


def _pre_call(xu3t, xi3t, eau3t, eai3t, ws, bs):
    big = pl.BlockSpec((L, D, BT), lambda i: (0, 0, i))
    nd = jax.ShapeDtypeStruct((N, D), jnp.float32)
    return pl.pallas_call(
        _pre_body,
        grid=(GRIDT,),
        in_specs=[big] * 4 + [_rep((D, D))] * 4 + [_rep((1, D))] * 4,
        out_specs=[pl.BlockSpec((BT, D), lambda i: (i, 0))] * 7,
        out_shape=[nd] * 7,
    )(xu3t, xi3t, eau3t, eai3t, *ws, *bs)


def _v_body(packed, *refs):
    if packed:
        (pk, vprev, w_ui, w_iu, cu, ci, du, di, vp_o) = refs
        xu = pk[:, :D] + vprev[:, D:]
        xi = pk[:, D:] + vprev[:, :D]
    else:
        (xur, xir, w_ui, w_iu, cu, ci, du, di, vp_o) = refs
        xu = xur[...]
        xi = xir[...]
    yu = lax.rsqrt(du[...] + 1.0) * (
        jnp.dot(xu, w_ui[...], preferred_element_type=jnp.float32) + cu[...])
    yi = lax.rsqrt(di[...] + 1.0) * (
        jnp.dot(xi, w_iu[...], preferred_element_type=jnp.float32) + ci[...])
    vp_o[...] = jnp.concatenate([yu, yi], axis=1)


def _v_call(packed, xs, w_ui, w_iu, cu, ci, du, di):
    blk = pl.BlockSpec((BN, D), lambda i: (i, 0))
    pk = pl.BlockSpec((BN, 2 * D), lambda i: (i, 0))
    deg = pl.BlockSpec((BN, 1), lambda i: (i, 0))
    x_specs = [pk, pk] if packed else [blk] * 2
    return pl.pallas_call(
        functools.partial(_v_body, packed),
        grid=(GRID,),
        in_specs=x_specs + [_rep((D, D))] * 2 + [blk] * 2 + [deg] * 2,
        out_specs=pk,
        out_shape=jax.ShapeDtypeStruct((N, 2 * D), jnp.float32),
    )(*xs, w_ui, w_iu, cu, ci, du, di)


def _fin_body(xu, xi, p1, p2, v0, v1, eau, wf1, bf1, wf2r, bf2, out):
    u = xu[...] + p1[:, :D] + v0[:, D:] + p2[:, :D] + v1[:, D:]
    it = xi[...] + p1[:, D:] + v0[:, :D] + p2[:, D:] + v1[:, :D]
    sv = (u + it) * (1.0 / 3.0) + 4.0 * eau[...]
    h = jnp.dot(sv, wf1[...], preferred_element_type=jnp.float32) + bf1[...]
    h = jnp.where(h >= 0, h, 0.01 * h)
    out[...] = jnp.sum(h * wf2r[...], axis=1, keepdims=True) + bf2[...]


def kernel(x_user, x_item, edge_attr_ui, edge_attr_iu,
           edge_index_ui, edge_index_iu,
           W1_ui0, b1_ui0, W2_ui0, b2_ui0, W3_ui0, b3_ui0,
           W1_iu0, b1_iu0, W2_iu0, b2_iu0, W3_iu0, b3_iu0,
           W1_ui1, b1_ui1, W2_ui1, b2_ui1, W3_ui1, b3_ui1,
           W1_iu1, b1_iu1, W2_iu1, b2_iu1, W3_iu1, b3_iu1,
           Wf1, bf1, Wf2, bf2):
    ei_ui = edge_index_ui.astype(jnp.int32)
    ei_iu = edge_index_iu.astype(jnp.int32)
    pad_r = jnp.zeros((EPAD - E,), jnp.int32)
    pad_c = jnp.full((EPAD - E,), N, jnp.int32)
    rows_ui = jnp.concatenate([ei_ui[0], pad_r]).reshape(NBT, BK)
    cols_ui = jnp.concatenate([ei_ui[1], pad_c]).reshape(NBT, BK)
    rows_iu = jnp.concatenate([ei_iu[0], pad_r]).reshape(NBT, BK)
    cols_iu = jnp.concatenate([ei_iu[1], pad_c]).reshape(NBT, BK)
    cols2 = jnp.stack([cols_ui, cols_iu])

    degc = _get_deg_kernel()(cols2)
    du = degc[0, :N, 0:1]
    di = degc[1, :N, 0:1]

    bsu0 = (b1_ui0 + b3_ui0).reshape(1, D)
    bsu1 = (b1_ui1 + b3_ui1).reshape(1, D)
    bsi0 = (b1_iu0 + b3_iu0).reshape(1, D)
    bsi1 = (b1_iu1 + b3_iu1).reshape(1, D)
    tr = lambda a: jnp.transpose(a, (1, 2, 0))
    xu, xi, eau, cu0, cu1, ci0, ci1 = _pre_call(
        tr(x_user), tr(x_item), tr(edge_attr_ui), tr(edge_attr_iu),
        (W3_ui0, W3_ui1, W3_iu0, W3_iu1), (bsu0, bsu1, bsi0, bsi1))

    agg = _get_agg_kernel()

    def agg_call(vpack):
        v2 = vpack.reshape(4 * N, H)
        return agg(v2, rows_ui, cols_ui, rows_iu, cols_iu)

    vp0 = _v_call(False, (xu, xi), W1_ui0, W1_iu0, cu0, ci0, du, di)
    pk1 = agg_call(vp0)

    vp1 = _v_call(True, (pk1, vp0), W1_ui1, W1_iu1, cu1, ci1, du, di)
    pk2 = agg_call(vp1)

    blk = pl.BlockSpec((BN, D), lambda i: (i, 0))
    pk = pl.BlockSpec((BN, 2 * D), lambda i: (i, 0))
    logits = pl.pallas_call(
        _fin_body,
        grid=(GRID,),
        in_specs=[blk, blk, pk, pk, pk, pk, blk, _rep((D, D)),
                  _rep((1, D)), _rep((1, D)), _rep((1, 1))],
        out_specs=pl.BlockSpec((BN, 1), lambda i: (i, 0)),
        out_shape=jax.ShapeDtypeStruct((N, 1), jnp.float32),
    )(xu, xi, pk1, pk2, vp0, vp1, eau,
      Wf1, bf1.reshape(1, D), Wf2.reshape(1, D), bf2.reshape(1, 1))
    return logits
